# baseline (speedup 1.0000x reference)
import functools

import jax
import jax.numpy as jnp
from jax import lax
from jax.experimental import pallas as pl
from jax.experimental.pallas import tpu as pltpu
from jax.experimental.pallas import tpu_sc as plsc

_EMBED = 64
_NC = 2
_NS = 16
_NW = _NC * _NS
_BBLK = 128
_DT = _EMBED // 8


@functools.lru_cache(maxsize=None)
def _build(batch: int, seq: int):
  assert batch == _NW * _BBLK
  mesh = plsc.VectorSubcoreMesh(
      core_axis_name="c", subcore_axis_name="s",
      num_cores=_NC, num_subcores=_NS)

  @functools.partial(
      pl.kernel,
      out_type=jax.ShapeDtypeStruct((seq, _DT, _NW, 8, _BBLK), jnp.float32),
      mesh=mesh,
      compiler_params=pltpu.CompilerParams(needs_layout_passes=False),
      scratch_types=[
          pltpu.VMEM((seq, _BBLK), jnp.int32),
          pltpu.VMEM((_BBLK, 2 * _EMBED), jnp.float32),
          pltpu.VMEM((_BBLK, 2 * _EMBED), jnp.float32),
          pltpu.VMEM((_BBLK, 2 * _EMBED), jnp.float32),
          pltpu.VMEM((_BBLK, 2 * _EMBED), jnp.float32),
          pltpu.VMEM((_DT, 8, _BBLK), jnp.float32),
          pltpu.VMEM((_DT, 8, _BBLK), jnp.float32),
          pltpu.VMEM((_DT, 8, _BBLK), jnp.float32),
          pltpu.VMEM((_DT, 8, _BBLK), jnp.float32),
          pltpu.SemaphoreType.DMA,
          pltpu.SemaphoreType.DMA,
          pltpu.SemaphoreType.DMA,
          pltpu.SemaphoreType.DMA,
          pltpu.SemaphoreType.DMA,
      ],
  )
  def emb(table_hbm, idx_hbm, out_hbm, idx_v, rows0, rows1, rows2, rows3,
          t0, t1, t2, t3, gsem, osem0, osem1, osem2, osem3):
    wid = lax.axis_index("s") * _NC + lax.axis_index("c")
    pltpu.sync_copy(idx_hbm.at[wid], idx_v)

    rbufs = (rows0, rows1, rows2, rows3)
    tbufs = (t0, t1, t2, t3)
    osems = (osem0, osem1, osem2, osem3)
    nbuf = 4

    def fire(s, p):
      return pltpu.async_copy(
          table_hbm.at[idx_v.at[s]], rbufs[p], gsem)

    def step(s, p, has_next, not_first):
      rows_buf, tbuf, osem = rbufs[p], tbufs[p], osems[p]
      pltpu.make_async_copy(
          table_hbm.at[idx_v.at[s]], rows_buf, gsem).wait()
      @pl.when(has_next)
      def _():
        fire(s + nbuf - 1, p - 1 if p else nbuf - 1)

      any_zero = None
      for i in range(_BBLK // 16):
        v = idx_v[s, pl.ds(i * 16, 16)]
        zm = v == 0
        any_zero = zm if any_zero is None else (any_zero | zm)

      @pl.when(plsc.all_reduce_population_count(any_zero)[0] > 0)
      def _():
        def fix_group(gi, carry):
          v = idx_v[s, pl.ds(gi * 16, 16)]
          zm = v == 0
          rowids = gi * 16 + lax.iota(jnp.int32, 16)

          @pl.when(plsc.all_reduce_population_count(zm)[0] > 0)
          def _():
            def fix_col(col, inner):
              plsc.store_scatter(
                  rows_buf,
                  [rowids, jnp.zeros((16,), jnp.int32) + col],
                  jnp.zeros((16,), jnp.float32),
                  mask=zm)
              return inner
            lax.fori_loop(0, _EMBED, fix_col, 0)
          return carry
        lax.fori_loop(0, _BBLK // 16, fix_group, 0)

      @pl.when(not_first)
      def _():
        for dt in range(_DT):
          pltpu.make_async_copy(
              tbuf.at[dt], out_hbm.at[0, dt, 0], osem).wait()

      def tr(d, carry):
        dt = d // 8
        dr = d - dt * 8
        col = jnp.zeros((16,), jnp.int32) + d
        for k in range(_BBLK // 16):
          rows = k * 16 + lax.iota(jnp.int32, 16)
          tbuf[dt, dr, pl.ds(k * 16, 16)] = plsc.load_gather(
              rows_buf, [rows, col])
        return carry
      lax.fori_loop(0, _EMBED, tr, 0)

      for dt in range(_DT):
        pltpu.async_copy(tbuf.at[dt], out_hbm.at[s, dt, wid], osem)

    for q in range(3):
      fire(q, q)
    @pl.loop(0, seq // 4)
    def _quads(s4):
      for p in range(4):
        s = 4 * s4 + p
        step(s, p, (s4 < seq // 4 - 1) | (p == 0), s4 > 0)

    for p in range(4):
      for dt in range(_DT):
        pltpu.make_async_copy(
            tbufs[p].at[dt], out_hbm.at[0, dt, 0], osems[p]).wait()

  return emb


@jax.jit
def kernel(table, input):
  b, s = input.shape
  table_p = jnp.pad(table, ((0, 0), (0, 2 * _EMBED - table.shape[1])))
  idx = input.astype(jnp.int32).reshape(_NW, _BBLK, s).transpose(0, 2, 1)
  out5 = _build(b, s)(table_p, idx)
  return out5.transpose(2, 4, 0, 1, 3).reshape(b, s, _EMBED)


# speedup vs baseline: 1.7525x; 1.7525x over previous
import functools

import jax
import jax.numpy as jnp
from jax import lax
from jax.experimental import pallas as pl
from jax.experimental.pallas import tpu as pltpu
from jax.experimental.pallas import tpu_sc as plsc

_EMBED = 64
_NC = 2
_NS = 16
_NW = _NC * _NS
_IDXW = 128
_NBUF = 4


@functools.lru_cache(maxsize=None)
def _build(rows_total: int):
  b_per_w = rows_total // _NW
  n_chunks = b_per_w // _IDXW
  mesh = plsc.VectorSubcoreMesh(
      core_axis_name="c", subcore_axis_name="s",
      num_cores=_NC, num_subcores=_NS)

  @functools.partial(
      pl.kernel,
      out_type=jax.ShapeDtypeStruct((rows_total, 2 * _EMBED), jnp.float32),
      mesh=mesh,
      compiler_params=pltpu.CompilerParams(needs_layout_passes=False),
      scratch_types=[
          pltpu.VMEM((n_chunks, _IDXW), jnp.int32),
          pltpu.VMEM((_IDXW, 2 * _EMBED), jnp.float32),
          pltpu.VMEM((_IDXW, 2 * _EMBED), jnp.float32),
          pltpu.VMEM((_IDXW, 2 * _EMBED), jnp.float32),
          pltpu.VMEM((_IDXW, 2 * _EMBED), jnp.float32),
          pltpu.SemaphoreType.DMA,
          pltpu.SemaphoreType.DMA,
          pltpu.SemaphoreType.DMA,
          pltpu.SemaphoreType.DMA,
          pltpu.SemaphoreType.DMA,
      ],
  )
  def emb(table_hbm, idx_hbm, out_hbm, idx_v, rows0, rows1, rows2, rows3,
          gsem, osem0, osem1, osem2, osem3):
    wid = lax.axis_index("s") * _NC + lax.axis_index("c")
    out_base = wid * b_per_w
    pltpu.sync_copy(idx_hbm.at[wid], idx_v)

    rbufs = (rows0, rows1, rows2, rows3)
    osems = (osem0, osem1, osem2, osem3)

    def fire(g, p):
      pltpu.async_copy(table_hbm.at[idx_v.at[g]], rbufs[p], gsem)

    def step(g, p, has_next, reused):
      rows_buf, osem = rbufs[p], osems[p]
      pltpu.make_async_copy(
          table_hbm.at[idx_v.at[g]], rows_buf, gsem).wait()

      np_ = (p + 2) % _NBUF

      @pl.when(reused & has_next)
      def _():
        pltpu.make_async_copy(
            rbufs[np_], out_hbm.at[pl.ds(0, _IDXW), :], osems[np_]).wait()

      @pl.when(has_next)
      def _():
        fire(g + 2, np_)

      any_zero = None
      for i in range(_IDXW // 16):
        v = idx_v[g, pl.ds(i * 16, 16)]
        zm = v == 0
        any_zero = zm if any_zero is None else (any_zero | zm)

      @pl.when(plsc.all_reduce_population_count(any_zero)[0] > 0)
      def _():
        def fix_group(gi, carry):
          v = idx_v[g, pl.ds(gi * 16, 16)]
          zm = v == 0
          rowids = gi * 16 + lax.iota(jnp.int32, 16)

          @pl.when(plsc.all_reduce_population_count(zm)[0] > 0)
          def _():
            def fix_col(col, inner):
              plsc.store_scatter(
                  rows_buf,
                  [rowids, jnp.zeros((16,), jnp.int32) + col],
                  jnp.zeros((16,), jnp.float32),
                  mask=zm)
              return inner
            lax.fori_loop(0, _EMBED, fix_col, 0)
          return carry
        lax.fori_loop(0, _IDXW // 16, fix_group, 0)

      pltpu.async_copy(
          rows_buf, out_hbm.at[pl.ds(out_base + g * _IDXW, _IDXW), :], osem)

    fire(0, 0)
    fire(1, 1)

    @pl.loop(0, n_chunks // _NBUF)
    def _rounds(g4):
      for p in range(_NBUF):
        g = _NBUF * g4 + p
        has_next = (g4 < n_chunks // _NBUF - 1) | (p < 2)
        reused = (g4 > 0) | (p >= 2)
        step(g, p, has_next, reused)

    for p in range(_NBUF):
      pltpu.make_async_copy(
          rbufs[p], out_hbm.at[pl.ds(0, _IDXW), :], osems[p]).wait()

  return emb


@jax.jit
def kernel(table, input):
  b, s = input.shape
  rows_total = b * s
  table_p = jnp.pad(table, ((0, 0), (0, 2 * _EMBED - table.shape[1])))
  idx = input.reshape(_NW, rows_total // (_NW * _IDXW), _IDXW).astype(
      jnp.int32)
  out = _build(rows_total)(table_p, idx)
  return out[:, :_EMBED].reshape(b, s, _EMBED)


# speedup vs baseline: 1.7545x; 1.0012x over previous
import functools

import jax
import jax.numpy as jnp
from jax import lax
from jax.experimental import pallas as pl
from jax.experimental.pallas import tpu as pltpu
from jax.experimental.pallas import tpu_sc as plsc

_EMBED = 64
_NC = 2
_NS = 16
_NW = _NC * _NS
_IDXW = 128
_NBUF = 4


@functools.lru_cache(maxsize=None)
def _build(rows_total: int):
  b_per_w = rows_total // _NW
  n_chunks = b_per_w // _IDXW
  mesh = plsc.VectorSubcoreMesh(
      core_axis_name="c", subcore_axis_name="s",
      num_cores=_NC, num_subcores=_NS)

  @functools.partial(
      pl.kernel,
      out_type=jax.ShapeDtypeStruct((rows_total, 2 * _EMBED), jnp.float32),
      mesh=mesh,
      compiler_params=pltpu.CompilerParams(
          needs_layout_passes=False, disable_bounds_checks=True,
          skip_device_barrier=True),
      scratch_types=[
          pltpu.VMEM((n_chunks, _IDXW), jnp.int32),
          pltpu.VMEM((_IDXW, 2 * _EMBED), jnp.float32),
          pltpu.VMEM((_IDXW, 2 * _EMBED), jnp.float32),
          pltpu.VMEM((_IDXW, 2 * _EMBED), jnp.float32),
          pltpu.VMEM((_IDXW, 2 * _EMBED), jnp.float32),
          pltpu.SemaphoreType.DMA,
          pltpu.SemaphoreType.DMA,
          pltpu.SemaphoreType.DMA,
          pltpu.SemaphoreType.DMA,
          pltpu.SemaphoreType.DMA,
      ],
  )
  def emb(table_hbm, idx_hbm, out_hbm, idx_v, rows0, rows1, rows2, rows3,
          gsem, osem0, osem1, osem2, osem3):
    wid = lax.axis_index("s") * _NC + lax.axis_index("c")
    out_base = wid * b_per_w
    pltpu.sync_copy(idx_hbm.at[wid], idx_v)

    rbufs = (rows0, rows1, rows2, rows3)
    osems = (osem0, osem1, osem2, osem3)

    def fire(g, p):
      pltpu.async_copy(table_hbm.at[idx_v.at[g]], rbufs[p], gsem)

    def step(g, p, has_next, reused):
      rows_buf, osem = rbufs[p], osems[p]
      pltpu.make_async_copy(
          table_hbm.at[idx_v.at[g]], rows_buf, gsem).wait()

      np_ = (p + 2) % _NBUF

      @pl.when(reused & has_next)
      def _():
        pltpu.make_async_copy(
            rbufs[np_], out_hbm.at[pl.ds(0, _IDXW), :], osems[np_]).wait()

      @pl.when(has_next)
      def _():
        fire(g + 2, np_)

      any_zero = None
      for i in range(_IDXW // 16):
        v = idx_v[g, pl.ds(i * 16, 16)]
        zm = v == 0
        any_zero = zm if any_zero is None else (any_zero | zm)

      @pl.when(plsc.all_reduce_population_count(any_zero)[0] > 0)
      def _():
        def fix_group(gi, carry):
          v = idx_v[g, pl.ds(gi * 16, 16)]
          zm = v == 0
          rowids = gi * 16 + lax.iota(jnp.int32, 16)

          @pl.when(plsc.all_reduce_population_count(zm)[0] > 0)
          def _():
            def fix_col(col, inner):
              plsc.store_scatter(
                  rows_buf,
                  [rowids, jnp.zeros((16,), jnp.int32) + col],
                  jnp.zeros((16,), jnp.float32),
                  mask=zm)
              return inner
            lax.fori_loop(0, _EMBED, fix_col, 0)
          return carry
        lax.fori_loop(0, _IDXW // 16, fix_group, 0)

      pltpu.async_copy(
          rows_buf, out_hbm.at[pl.ds(out_base + g * _IDXW, _IDXW), :], osem)

    fire(0, 0)
    fire(1, 1)

    @pl.loop(0, n_chunks // _NBUF)
    def _rounds(g4):
      for p in range(_NBUF):
        g = _NBUF * g4 + p
        has_next = (g4 < n_chunks // _NBUF - 1) | (p < 2)
        reused = (g4 > 0) | (p >= 2)
        step(g, p, has_next, reused)

    for p in range(_NBUF):
      pltpu.make_async_copy(
          rbufs[p], out_hbm.at[pl.ds(0, _IDXW), :], osems[p]).wait()

  return emb


@jax.jit
def kernel(table, input):
  b, s = input.shape
  rows_total = b * s
  table_p = jnp.pad(table, ((0, 0), (0, 2 * _EMBED - table.shape[1])))
  idx = input.reshape(_NW, rows_total // (_NW * _IDXW), _IDXW).astype(
      jnp.int32)
  out = _build(rows_total)(table_p, idx)
  return out[:, :_EMBED].reshape(b, s, _EMBED)
